# baseline (speedup 1.0000x reference)
import numpy as np

import jax
import jax.numpy as jnp
from jax.experimental import pallas as pl
from jax.experimental.pallas import tpu as pltpu

_N = 39
_NP = _N * _N
_NP_PAD = 1536
_TB = 2048
_VMEM_BYTES = 40 << 20


def _pair_diff_body(r_ref, d_ref, o_ref):
    r16 = r_ref[...].astype(jnp.bfloat16)
    acc = jax.lax.dot_general(
        r16, d_ref[...],
        dimension_numbers=(((1,), (0,)), ((), ())),
        preferred_element_type=jnp.float32,
    )
    o_ref[...] = acc.astype(jnp.bfloat16)


def _pair_diff_const() -> np.ndarray:
    eye = np.eye(_N, dtype=np.float32)
    d = (eye[:, :, None] - eye[:, None, :]).reshape(_N, _NP)
    d_pad = np.zeros((_N, _NP_PAD), dtype=np.float32)
    d_pad[:, :_NP] = d
    return d_pad.astype(np.dtype("bfloat16"))


def kernel(batch_ranking, batch_label):
    del batch_label
    r = jnp.asarray(batch_ranking, jnp.float32).reshape(-1, _N)
    b_total = r.shape[0]

    tb = min(_TB, b_total)
    if b_total >= 16:
        half = -(-b_total // 2)
        tb = min(tb, ((half + 7) // 8) * 8)
    grid = (pl.cdiv(b_total, tb),)

    d = jnp.asarray(_pair_diff_const())

    out = pl.pallas_call(
        _pair_diff_body,
        out_shape=jax.ShapeDtypeStruct((b_total, _NP_PAD), jnp.bfloat16),
        grid=grid,
        in_specs=[
            pl.BlockSpec((tb, _N), lambda i: (i, 0)),
            pl.BlockSpec((_N, _NP_PAD), lambda i: (0, 0)),
        ],
        out_specs=pl.BlockSpec((tb, _NP_PAD), lambda i: (i, 0)),
        compiler_params=pltpu.CompilerParams(
            dimension_semantics=("parallel",),
            vmem_limit_bytes=_VMEM_BYTES,
        ),
        cost_estimate=pl.CostEstimate(
            flops=2 * b_total * _N * _NP_PAD,
            transcendentals=0,
            bytes_accessed=b_total * _N * 4 + _N * _NP_PAD * 2
            + b_total * _NP_PAD * 2,
        ),
    )(r, d)

    return out[:, :_NP].astype(jnp.float32).reshape(b_total, _N, _N)


# speedup vs baseline: 1.0877x; 1.0877x over previous
import numpy as np

import jax
import jax.numpy as jnp
from jax.experimental import pallas as pl
from jax.experimental.pallas import tpu as pltpu

_N = 39
_NP = _N * _N
_NP_PAD = 1536
_TB = 1024
_K = 4
_VMEM_BYTES = 40 << 20


def _pair_diff_const() -> np.ndarray:
    eye = np.eye(_N, dtype=np.float32)
    d = (eye[:, :, None] - eye[:, None, :]).reshape(_N, _NP)
    d_pad = np.zeros((_N, _NP_PAD), dtype=np.float32)
    d_pad[:, :_NP] = d
    return d_pad.astype(np.dtype("bfloat16"))


def _matmul_block(r_ref, d_ref):
    r16 = r_ref[...].astype(jnp.bfloat16)
    return jax.lax.dot_general(
        r16, d_ref[...],
        dimension_numbers=(((1,), (0,)), ((), ())),
        preferred_element_type=jnp.float32,
    )


def _manual_body(r_ref, d_ref, o_hbm, scratch, sems):
    i = pl.program_id(0)
    n = pl.num_programs(0)
    slot = jax.lax.rem(i, _K)

    def _out_copy(step, s):
        return pltpu.make_async_copy(
            scratch.at[s],
            o_hbm.at[pl.ds(step * _TB, _TB), :],
            sems.at[s],
        )

    @pl.when(i >= _K)
    def _():
        _out_copy(i - _K, slot).wait()

    scratch[slot] = _matmul_block(r_ref, d_ref)[:, :_NP]
    _out_copy(i, slot).start()

    @pl.when(i == n - 1)
    def _():
        for k in range(_K):
            step = i - (_K - 1) + k
            _out_copy(step, jax.lax.rem(step, _K)).wait()


def _simple_body(r_ref, d_ref, o_ref):
    o_ref[...] = _matmul_block(r_ref, d_ref)


def kernel(batch_ranking, batch_label):
    del batch_label
    r = jnp.asarray(batch_ranking, jnp.float32).reshape(-1, _N)
    b_total = r.shape[0]
    d = jnp.asarray(_pair_diff_const())

    if b_total % _TB == 0 and (b_total // _TB) >= _K:
        grid = (b_total // _TB,)
        out = pl.pallas_call(
            _manual_body,
            out_shape=jax.ShapeDtypeStruct((b_total, _NP), jnp.float32),
            grid=grid,
            in_specs=[
                pl.BlockSpec((_TB, _N), lambda i: (i, 0)),
                pl.BlockSpec((_N, _NP_PAD), lambda i: (0, 0)),
            ],
            out_specs=pl.BlockSpec(memory_space=pl.ANY),
            scratch_shapes=[
                pltpu.VMEM((_K, _TB, _NP), jnp.float32),
                pltpu.SemaphoreType.DMA((_K,)),
            ],
            compiler_params=pltpu.CompilerParams(
                dimension_semantics=("arbitrary",),
                vmem_limit_bytes=_VMEM_BYTES,
            ),
            cost_estimate=pl.CostEstimate(
                flops=2 * b_total * _N * _NP_PAD,
                transcendentals=0,
                bytes_accessed=b_total * _N * 4 + _N * _NP_PAD * 2
                + b_total * _NP * 4,
            ),
        )(r, d)
        return out.reshape(b_total, _N, _N)

    tb = min(_TB, b_total)
    if b_total >= 16:
        half = -(-b_total // 2)
        tb = min(tb, ((half + 7) // 8) * 8)
    grid = (pl.cdiv(b_total, tb),)
    out = pl.pallas_call(
        _simple_body,
        out_shape=jax.ShapeDtypeStruct((b_total, _NP_PAD), jnp.float32),
        grid=grid,
        in_specs=[
            pl.BlockSpec((tb, _N), lambda i: (i, 0)),
            pl.BlockSpec((_N, _NP_PAD), lambda i: (0, 0)),
        ],
        out_specs=pl.BlockSpec((tb, _NP_PAD), lambda i: (i, 0)),
        compiler_params=pltpu.CompilerParams(
            dimension_semantics=("parallel",),
            vmem_limit_bytes=_VMEM_BYTES,
        ),
    )(r, d)
    return out[:, :_NP].reshape(b_total, _N, _N)
